# baseline (speedup 1.0000x reference)
import functools

import jax
import jax.numpy as jnp
from jax import lax
from jax.experimental import pallas as pl
from jax.experimental.pallas import tpu as pltpu
from jax.experimental.pallas import tpu_sc as plsc


def _harmgram_sc(spec_flat, idx_flat, R, K, F):
    info = plsc.get_sparse_core_info()
    nc, ns = info.num_cores, info.num_subcores
    nw = nc * ns
    rows_w = R // nw
    kv = K // 16

    cr = 20
    nchk = rows_w // cr
    bufw = cr * F
    obw = cr * K

    mesh = plsc.VectorSubcoreMesh(core_axis_name="c", subcore_axis_name="s")

    @functools.partial(
        pl.kernel,
        mesh=mesh,
        compiler_params=pltpu.CompilerParams(needs_layout_passes=False),
        out_type=jax.ShapeDtypeStruct((R * K,), jnp.float32),
        scratch_types=[
            pltpu.VMEM((K,), jnp.int32),
            pltpu.VMEM((bufw,), jnp.float32),
            pltpu.VMEM((bufw,), jnp.float32),
            pltpu.VMEM((obw,), jnp.float32),
            pltpu.VMEM((obw,), jnp.float32),
            pltpu.SemaphoreType.DMA,
            pltpu.SemaphoreType.DMA,
            pltpu.SemaphoreType.DMA,
            pltpu.SemaphoreType.DMA,
        ],
    )
    def run(spec_hbm, idx_hbm, out_hbm, base_v, in0, in1, ob0, ob1,
            si0, si1, so0, so1):
        wid = lax.axis_index("s") * nc + lax.axis_index("c")
        row0 = wid * rows_w
        pltpu.sync_copy(idx_hbm, base_v)

        ibufs, obufs = (in0, in1), (ob0, ob1)
        isems, osems = (si0, si1), (so0, so1)

        def start_in(g):
            off = (row0 + g * cr) * F
            return pltpu.async_copy(
                spec_hbm.at[pl.ds(off, bufw)], ibufs[g % 2], isems[g % 2])

        def start_out(g):
            off = (row0 + g * cr) * K
            return pltpu.async_copy(
                obufs[g % 2], out_hbm.at[pl.ds(off, obw)], osems[g % 2])

        in_cp = {0: start_in(0)}
        out_cp = {}
        for g in range(nchk):
            if g + 1 < nchk:
                in_cp[g + 1] = start_in(g + 1)
            in_cp.pop(g).wait()
            if g >= 2:
                out_cp.pop(g - 2).wait()

            buf, ob = ibufs[g % 2], obufs[g % 2]
            carry = [base_v[pl.ds(16 * j, 16)] for j in range(kv)]

            def body(r, c, buf=buf, ob=ob):
                for j in range(kv):
                    ob[pl.ds(r * K + 16 * j, 16)] = plsc.load_gather(
                        buf, [c[j]])
                return [v + F for v in c]

            lax.fori_loop(0, cr, body, carry)
            out_cp[g] = start_out(g)

        for g in sorted(out_cp):
            out_cp.pop(g).wait()

    return run(spec_flat, idx_flat)


def kernel(specgram, hargram_idx):
    B, T, F = specgram.shape
    P, H = hargram_idx.shape
    spec_flat = specgram.reshape(-1)
    idx_flat = hargram_idx.reshape(-1).astype(jnp.int32)
    out = _harmgram_sc(spec_flat, idx_flat, B * T, P * H, F)
    return out.reshape(B, T, P, H)


# speedup vs baseline: 1.9182x; 1.9182x over previous
import functools

import jax
import jax.numpy as jnp
from jax import lax
from jax.experimental import pallas as pl
from jax.experimental.pallas import tpu as pltpu
from jax.experimental.pallas import tpu_sc as plsc


def _harmgram_sc(spec3d, idx_flat, B, T, P, H, F):
    K = P * H
    kv = K // 16
    U = (B * T) // 8
    upb = T // 8

    info = plsc.get_sparse_core_info()
    nc, ns = info.num_cores, info.num_subcores
    nw = nc * ns
    ng_small = U // nw
    big = U - ng_small * nw
    pairs = ng_small // 2

    mesh = plsc.VectorSubcoreMesh(core_axis_name="c", subcore_axis_name="s")

    @functools.partial(
        pl.kernel,
        mesh=mesh,
        compiler_params=pltpu.CompilerParams(needs_layout_passes=False),
        out_type=jax.ShapeDtypeStruct((B, T, P, H), jnp.float32),
        scratch_types=[
            pltpu.VMEM((K,), jnp.int32),
            pltpu.VMEM((8, F), jnp.float32),
            pltpu.VMEM((8, F), jnp.float32),
            pltpu.VMEM((8 * P, H), jnp.float32),
            pltpu.VMEM((8 * P, H), jnp.float32),
            pltpu.SemaphoreType.DMA,
            pltpu.SemaphoreType.DMA,
            pltpu.SemaphoreType.DMA,
            pltpu.SemaphoreType.DMA,
        ],
    )
    def run(spec_hbm, idx_hbm, out_hbm, base_v, in0, in1, ob0, ob1,
            si0, si1, so0, so1):
        wid = lax.axis_index("s") * nc + lax.axis_index("c")
        is_big = wid < big
        u0 = ng_small * wid + jnp.minimum(wid, big)
        u1 = u0 + ng_small + is_big.astype(jnp.int32)
        pltpu.sync_copy(idx_hbm, base_v)

        def start_in(u, buf, sem):
            return pltpu.async_copy(spec_hbm.at[u], buf, sem)

        def start_out(u, ob, sem):
            b = u // upb
            t0 = (u - b * upb) * 8
            for r in range(8):
                pltpu.async_copy(
                    ob.at[pl.ds(r * P, P), :],
                    out_hbm.at[b, t0 + r], sem)

        def drain_out(ob, sem):
            for r in range(8):
                pltpu.make_async_copy(
                    ob.at[pl.ds(r * P, P), :],
                    out_hbm.at[0, 0], sem).wait()

        def compute(buf, ob):
            def body(r, _):
                rvec = jnp.full((16,), r, jnp.int32)
                for p in range(kv):
                    ob[r * P + p, :] = plsc.load_gather(
                        buf, [rvec, base_v[pl.ds(16 * p, 16)]])
                return 0

            lax.fori_loop(0, 8, body, 0)

        start_in(u0, in0, si0)

        def pair_body(i, _):
            ua = u0 + 2 * i
            start_in(ua + 1, in1, si1)
            pltpu.make_async_copy(spec_hbm.at[ua], in0, si0).wait()

            @pl.when(i > 0)
            def _():
                drain_out(ob0, so0)

            compute(in0, ob0)
            start_out(ua, ob0, so0)

            @pl.when(ua + 2 < u1)
            def _():
                start_in(ua + 2, in0, si0)

            pltpu.make_async_copy(spec_hbm.at[ua], in1, si1).wait()

            @pl.when(i > 0)
            def _():
                drain_out(ob1, so1)

            compute(in1, ob1)
            start_out(ua + 1, ob1, so1)
            return 0

        lax.fori_loop(0, pairs, pair_body, 0)

        @pl.when(is_big)
        def _():
            pltpu.make_async_copy(spec_hbm.at[u0], in0, si0).wait()
            drain_out(ob0, so0)
            compute(in0, ob0)
            start_out(u1 - 1, ob0, so0)

        drain_out(ob0, so0)
        drain_out(ob1, so1)

    return run(spec3d, idx_flat)


def kernel(specgram, hargram_idx):
    B, T, F = specgram.shape
    P, H = hargram_idx.shape
    spec3d = specgram.reshape((B * T) // 8, 8, F)
    idx_flat = hargram_idx.reshape(-1).astype(jnp.int32)
    return _harmgram_sc(spec3d, idx_flat, B, T, P, H, F)


# speedup vs baseline: 2.4230x; 1.2632x over previous
import functools

import jax
import jax.numpy as jnp
from jax import lax
from jax.experimental import pallas as pl
from jax.experimental.pallas import tpu as pltpu
from jax.experimental.pallas import tpu_sc as plsc


def _harmgram_sc(spec3d, idx_flat, B, T, P, H, F):
    K = P * H
    kv = K // 16
    U = (B * T) // 8
    upb = T // 8
    nt = F // 128

    info = plsc.get_sparse_core_info()
    nc, ns = info.num_cores, info.num_subcores
    nw = nc * ns
    ng_small = U // nw
    big = U - ng_small * nw
    pairs = ng_small // 2

    mesh = plsc.VectorSubcoreMesh(core_axis_name="c", subcore_axis_name="s")

    @functools.partial(
        pl.kernel,
        mesh=mesh,
        compiler_params=pltpu.CompilerParams(needs_layout_passes=False),
        out_type=jax.ShapeDtypeStruct((B, T, P, H), jnp.float32),
        scratch_types=[
            pltpu.VMEM((K,), jnp.int32),
            pltpu.VMEM((16,), jnp.int32),
            pltpu.VMEM((nt * 8, 128), jnp.float32),
            pltpu.VMEM((nt * 8, 128), jnp.float32),
            pltpu.VMEM((8 * P, H), jnp.float32),
            pltpu.VMEM((8 * P, H), jnp.float32),
            pltpu.SemaphoreType.DMA,
            pltpu.SemaphoreType.DMA,
            pltpu.SemaphoreType.DMA,
            pltpu.SemaphoreType.DMA,
        ],
    )
    def run(spec_hbm, idx_hbm, out_hbm, base_v, slots_v, in0, in1, ob0, ob1,
            si0, si1, so0, so1):
        wid = lax.axis_index("s") * nc + lax.axis_index("c")
        is_big = wid < big
        u0 = ng_small * wid + jnp.minimum(wid, big)
        u1 = u0 + ng_small + is_big.astype(jnp.int32)
        pltpu.sync_copy(idx_hbm, base_v)

        idx_vecs = [base_v[pl.ds(16 * p, 16)] for p in range(kv)]
        tile_vecs = [v >> 7 for v in idx_vecs]
        lane_vecs = [v & 127 for v in idx_vecs]
        one = jnp.full((16,), 1, jnp.int32)
        zero16 = jnp.full((16,), 0, jnp.int32)
        slots_v[...] = zero16
        for tv in tile_vecs:
            plsc.addupdate_scatter(slots_v, [tv], one)
        bits = jnp.where(slots_v[...] > 0, one, zero16)
        incl = plsc.cumsum(bits)
        excl = incl - bits
        slots_v[...] = excl
        slot_vecs = [plsc.load_gather(slots_v, [tv]) for tv in tile_vecs]
        rowbase = [sv * 8 for sv in slot_vecs]

        def fetch(u, buf, sem):
            for j in range(nt):
                @pl.when(bits[j] == 1)
                def _(j=j):
                    slot = excl[j]
                    pltpu.async_copy(
                        spec_hbm.at[u, :, pl.ds(128 * j, 128)],
                        buf.at[pl.ds(slot * 8, 8), :], sem)

        def drain_in(buf, sem):
            for j in range(nt):
                @pl.when(bits[j] == 1)
                def _():
                    pltpu.make_async_copy(
                        spec_hbm.at[0, :, pl.ds(0, 128)],
                        buf.at[pl.ds(0, 8), :], sem).wait()

        def start_out(u, ob, sem):
            b = u // upb
            t0 = (u - b * upb) * 8
            for r in range(8):
                pltpu.async_copy(
                    ob.at[pl.ds(r * P, P), :],
                    out_hbm.at[b, t0 + r], sem)

        def drain_out(ob, sem):
            for r in range(8):
                pltpu.make_async_copy(
                    ob.at[pl.ds(r * P, P), :],
                    out_hbm.at[0, 0], sem).wait()

        def compute(buf, ob):
            def body(r, _):
                for p in range(kv):
                    ob[r * P + p, :] = plsc.load_gather(
                        buf, [rowbase[p] + r, lane_vecs[p]])
                return 0

            lax.fori_loop(0, 8, body, 0)

        fetch(u0, in0, si0)

        def pair_body(i, _):
            ua = u0 + 2 * i
            fetch(ua + 1, in1, si1)
            drain_in(in0, si0)

            @pl.when(i > 0)
            def _():
                drain_out(ob0, so0)

            compute(in0, ob0)
            start_out(ua, ob0, so0)

            @pl.when(ua + 2 < u1)
            def _():
                fetch(ua + 2, in0, si0)

            drain_in(in1, si1)

            @pl.when(i > 0)
            def _():
                drain_out(ob1, so1)

            compute(in1, ob1)
            start_out(ua + 1, ob1, so1)
            return 0

        lax.fori_loop(0, pairs, pair_body, 0)

        @pl.when(is_big)
        def _():
            drain_in(in0, si0)
            drain_out(ob0, so0)
            compute(in0, ob0)
            start_out(u1 - 1, ob0, so0)

        drain_out(ob0, so0)
        drain_out(ob1, so1)

    return run(spec3d, idx_flat)


def kernel(specgram, hargram_idx):
    B, T, F = specgram.shape
    P, H = hargram_idx.shape
    spec3d = specgram.reshape((B * T) // 8, 8, F)
    idx_flat = hargram_idx.reshape(-1).astype(jnp.int32)
    return _harmgram_sc(spec3d, idx_flat, B, T, P, H, F)


# speedup vs baseline: 2.4726x; 1.0204x over previous
import functools

import jax
import jax.numpy as jnp
from jax import lax
from jax.experimental import pallas as pl
from jax.experimental.pallas import tpu as pltpu
from jax.experimental.pallas import tpu_sc as plsc


def _harmgram_sc(spec3d, idx_flat, B, T, P, H, F):
    K = P * H
    kv = K // 16
    U = (B * T) // 8
    upb = T // 8
    nt = F // 512

    info = plsc.get_sparse_core_info()
    nc, ns = info.num_cores, info.num_subcores
    nw = nc * ns
    ng_small = U // nw
    big = U - ng_small * nw
    pairs = ng_small // 2

    mesh = plsc.VectorSubcoreMesh(core_axis_name="c", subcore_axis_name="s")

    @functools.partial(
        pl.kernel,
        mesh=mesh,
        compiler_params=pltpu.CompilerParams(needs_layout_passes=False),
        out_type=jax.ShapeDtypeStruct((B, T, P, H), jnp.float32),
        scratch_types=[
            pltpu.VMEM((K,), jnp.int32),
            pltpu.VMEM((16,), jnp.int32),
            pltpu.VMEM((nt * 8, 512), jnp.float32),
            pltpu.VMEM((nt * 8, 512), jnp.float32),
            pltpu.VMEM((8 * P, H), jnp.float32),
            pltpu.VMEM((8 * P, H), jnp.float32),
            pltpu.SemaphoreType.DMA,
            pltpu.SemaphoreType.DMA,
            pltpu.SemaphoreType.DMA,
            pltpu.SemaphoreType.DMA,
        ],
    )
    def run(spec_hbm, idx_hbm, out_hbm, base_v, slots_v, in0, in1, ob0, ob1,
            si0, si1, so0, so1):
        wid = lax.axis_index("s") * nc + lax.axis_index("c")
        is_big = wid < big
        u0 = ng_small * wid + jnp.minimum(wid, big)
        u1 = u0 + ng_small + is_big.astype(jnp.int32)
        pltpu.sync_copy(idx_hbm, base_v)

        idx_vecs = [base_v[pl.ds(16 * p, 16)] for p in range(kv)]
        tile_vecs = [v >> 9 for v in idx_vecs]
        lane_vecs = [v & 511 for v in idx_vecs]
        one = jnp.full((16,), 1, jnp.int32)
        zero16 = jnp.full((16,), 0, jnp.int32)
        slots_v[...] = zero16
        for tv in tile_vecs:
            plsc.addupdate_scatter(slots_v, [tv], one)
        bits = jnp.where(slots_v[...] > 0, one, zero16)
        incl = plsc.cumsum(bits)
        excl = incl - bits
        slots_v[...] = excl
        slot_vecs = [plsc.load_gather(slots_v, [tv]) for tv in tile_vecs]
        rowbase = [sv * 8 for sv in slot_vecs]

        def fetch(u, buf, sem):
            for j in range(nt):
                @pl.when(bits[j] == 1)
                def _(j=j):
                    slot = excl[j]
                    pltpu.async_copy(
                        spec_hbm.at[u, :, pl.ds(512 * j, 512)],
                        buf.at[pl.ds(slot * 8, 8), :], sem)

        def drain_in(buf, sem):
            for j in range(nt):
                @pl.when(bits[j] == 1)
                def _():
                    pltpu.make_async_copy(
                        spec_hbm.at[0, :, pl.ds(0, 512)],
                        buf.at[pl.ds(0, 8), :], sem).wait()

        def start_out(u, ob, sem):
            b = u // upb
            t0 = (u - b * upb) * 8
            for r in range(8):
                pltpu.async_copy(
                    ob.at[pl.ds(r * P, P), :],
                    out_hbm.at[b, t0 + r], sem)

        def drain_out(ob, sem):
            for r in range(8):
                pltpu.make_async_copy(
                    ob.at[pl.ds(r * P, P), :],
                    out_hbm.at[0, 0], sem).wait()

        def compute(buf, ob):
            def body(r, _):
                for p in range(kv):
                    ob[r * P + p, :] = plsc.load_gather(
                        buf, [rowbase[p] + r, lane_vecs[p]])
                return 0

            lax.fori_loop(0, 8, body, 0)

        fetch(u0, in0, si0)

        def pair_body(i, _):
            ua = u0 + 2 * i
            fetch(ua + 1, in1, si1)
            drain_in(in0, si0)

            @pl.when(i > 0)
            def _():
                drain_out(ob0, so0)

            compute(in0, ob0)
            start_out(ua, ob0, so0)

            @pl.when(ua + 2 < u1)
            def _():
                fetch(ua + 2, in0, si0)

            drain_in(in1, si1)

            @pl.when(i > 0)
            def _():
                drain_out(ob1, so1)

            compute(in1, ob1)
            start_out(ua + 1, ob1, so1)
            return 0

        lax.fori_loop(0, pairs, pair_body, 0)

        @pl.when(is_big)
        def _():
            drain_in(in0, si0)
            drain_out(ob0, so0)
            compute(in0, ob0)
            start_out(u1 - 1, ob0, so0)

        drain_out(ob0, so0)
        drain_out(ob1, so1)

    return run(spec3d, idx_flat)


def kernel(specgram, hargram_idx):
    B, T, F = specgram.shape
    P, H = hargram_idx.shape
    spec3d = specgram.reshape((B * T) // 8, 8, F)
    idx_flat = hargram_idx.reshape(-1).astype(jnp.int32)
    return _harmgram_sc(spec3d, idx_flat, B, T, P, H, F)


# speedup vs baseline: 2.8257x; 1.1428x over previous
import functools

import jax
import jax.numpy as jnp
from jax import lax
from jax.experimental import pallas as pl
from jax.experimental.pallas import tpu as pltpu
from jax.experimental.pallas import tpu_sc as plsc


def _harmgram_sc(spec3d, idx_flat, B, T, P, H, F):
    K = P * H
    kv = K // 16
    U = (B * T) // 8
    upb = T // 8
    nt = F // 512

    info = plsc.get_sparse_core_info()
    nc, ns = info.num_cores, info.num_subcores
    nw = nc * ns
    ng_small = U // nw
    big = U - ng_small * nw
    pairs = ng_small // 2

    mesh = plsc.VectorSubcoreMesh(core_axis_name="c", subcore_axis_name="s")

    @functools.partial(
        pl.kernel,
        mesh=mesh,
        compiler_params=pltpu.CompilerParams(needs_layout_passes=False),
        out_type=jax.ShapeDtypeStruct((B, T, P, H), jnp.float32),
        scratch_types=[
            pltpu.VMEM((K,), jnp.int32),
            pltpu.VMEM((16,), jnp.int32),
            pltpu.VMEM((nt * 8, 512), jnp.float32),
            pltpu.VMEM((nt * 8, 512), jnp.float32),
            pltpu.VMEM((nt * 8, 512), jnp.float32),
            pltpu.VMEM((nt * 8, 512), jnp.float32),
            pltpu.VMEM((8 * P, H), jnp.float32),
            pltpu.VMEM((8 * P, H), jnp.float32),
            pltpu.VMEM((8 * P, H), jnp.float32),
            pltpu.VMEM((8 * P, H), jnp.float32),
            pltpu.SemaphoreType.DMA,
            pltpu.SemaphoreType.DMA,
            pltpu.SemaphoreType.DMA,
            pltpu.SemaphoreType.DMA,
            pltpu.SemaphoreType.DMA,
            pltpu.SemaphoreType.DMA,
            pltpu.SemaphoreType.DMA,
            pltpu.SemaphoreType.DMA,
        ],
    )
    def run(spec_hbm, idx_hbm, out_hbm, base_v, slots_v,
            in0, in1, in2, in3, ob0, ob1, ob2, ob3,
            si0, si1, si2, si3, so0, so1, so2, so3):
        ibufs, obufs = (in0, in1, in2, in3), (ob0, ob1, ob2, ob3)
        isems, osems = (si0, si1, si2, si3), (so0, so1, so2, so3)
        wid = lax.axis_index("s") * nc + lax.axis_index("c")
        is_big = wid < big
        u0 = ng_small * wid + jnp.minimum(wid, big)
        u1 = u0 + ng_small + is_big.astype(jnp.int32)
        pltpu.sync_copy(idx_hbm, base_v)

        idx_vecs = [base_v[pl.ds(16 * p, 16)] for p in range(kv)]
        tile_vecs = [v >> 9 for v in idx_vecs]
        lane_vecs = [v & 511 for v in idx_vecs]
        one = jnp.full((16,), 1, jnp.int32)
        zero16 = jnp.full((16,), 0, jnp.int32)
        slots_v[...] = zero16
        for tv in tile_vecs:
            plsc.addupdate_scatter(slots_v, [tv], one)
        bits = jnp.where(slots_v[...] > 0, one, zero16)
        incl = plsc.cumsum(bits)
        excl = incl - bits
        slots_v[...] = excl
        slot_vecs = [plsc.load_gather(slots_v, [tv]) for tv in tile_vecs]
        rowbase = [sv * 8 for sv in slot_vecs]

        def fetch(u, buf, sem):
            for j in range(nt):
                @pl.when(bits[j] == 1)
                def _(j=j):
                    slot = excl[j]
                    pltpu.async_copy(
                        spec_hbm.at[u, :, pl.ds(512 * j, 512)],
                        buf.at[pl.ds(slot * 8, 8), :], sem)

        def drain_in(buf, sem):
            for j in range(nt):
                @pl.when(bits[j] == 1)
                def _():
                    pltpu.make_async_copy(
                        spec_hbm.at[0, :, pl.ds(0, 512)],
                        buf.at[pl.ds(0, 8), :], sem).wait()

        def start_out(b, t0, ob, sem):
            for r in range(8):
                pltpu.async_copy(
                    ob.at[pl.ds(r * P, P), :],
                    out_hbm.at[b, t0 + r], sem)

        def drain_out(ob, sem):
            for r in range(8):
                pltpu.make_async_copy(
                    ob.at[pl.ds(r * P, P), :],
                    out_hbm.at[0, 0], sem).wait()

        def compute(buf, ob):
            def body(r, _):
                for p in range(kv):
                    ob[r * P + p, :] = plsc.load_gather(
                        buf, [rowbase[p] + r, lane_vecs[p]])
                return 0

            lax.fori_loop(0, 8, body, 0)

        for s in range(3):
            fetch(u0 + s, ibufs[s], isems[s])

        def unit_step(q, s, b, t0):
            @pl.when(q + 3 < u1)
            def _():
                fetch(q + 3, ibufs[(s + 3) % 4], isems[(s + 3) % 4])

            drain_in(ibufs[s], isems[s])

            @pl.when(q >= u0 + 4)
            def _():
                drain_out(obufs[s], osems[s])

            compute(ibufs[s], obufs[s])
            start_out(b, t0, obufs[s], osems[s])
            t1 = t0 + 8
            wrap = t1 >= T
            return (jnp.where(wrap, b + 1, b),
                    jnp.where(wrap, 0, t1))

        b_init = u0 // upb
        t_init = (u0 - b_init * upb) * 8

        def quad_body(i, bt):
            b, t0 = bt
            q = u0 + 4 * i
            for s in range(4):
                b, t0 = unit_step(q + s, s, b, t0)
            return (b, t0)

        quads = ng_small // 4
        b, t0 = lax.fori_loop(0, quads, quad_body, (b_init, t_init))

        qt = u0 + 4 * quads
        b, t0 = unit_step(qt, 0, b, t0)
        b, t0 = unit_step(qt + 1, 1, b, t0)

        @pl.when(is_big)
        def _():
            unit_step(qt + 2, 2, b, t0)

        for s in range(4):
            drain_out(obufs[s], osems[s])

    return run(spec3d, idx_flat)


def kernel(specgram, hargram_idx):
    B, T, F = specgram.shape
    P, H = hargram_idx.shape
    spec3d = specgram.reshape((B * T) // 8, 8, F)
    idx_flat = hargram_idx.reshape(-1).astype(jnp.int32)
    return _harmgram_sc(spec3d, idx_flat, B, T, P, H, F)
